# baseline (speedup 1.0000x reference)
import functools

import jax
import jax.numpy as jnp
from jax import lax
from jax.experimental import pallas as pl
from jax.experimental.pallas import tpu as pltpu
from jax.experimental.pallas import tpu_sc as plsc

B = 16384
G = 50
D = 64
C_PAD = 128
T_COL = 100
T_ROWS = 1024
NW = 32
RB = B // NW
N_GRP = RB // 16
N_CHUNK = 4


def _sc_body(t_hbm, genres_hbm, counts_hbm, t_v, gen_v, counts_v, gsem, csem):
    wid = lax.axis_index("s") * 2 + lax.axis_index("c")
    base = wid * RB

    gen_cp = pltpu.async_copy(genres_hbm.at[:, pl.ds(base, RB)], gen_v, gsem)
    pltpu.sync_copy(t_hbm.at[pl.ds(base, RB)], t_v)

    iota16 = lax.broadcasted_iota(jnp.int32, (16,), 0)
    ones16 = jnp.ones((16,), jnp.float32)
    zeros16 = jnp.zeros((16,), jnp.float32)
    tcol16 = jnp.full((16,), T_COL, jnp.int32)

    def zrow(r, carry):
        for c in range(C_PAD // 16):
            counts_v[r, pl.ds(c * 16, 16)] = zeros16
        return carry

    lax.fori_loop(0, RB, zrow, 0)
    gen_cp.wait()

    grp_per_chunk = N_GRP // N_CHUNK
    rows_per_chunk = RB // N_CHUNK

    def grp(i, carry):
        rows16 = i * 16 + iota16
        for s in range(G):
            g16 = gen_v[s, pl.ds(i * 16, 16)]
            plsc.addupdate_scatter(counts_v, [rows16, g16], ones16)
        t16 = t_v[pl.ds(i * 16, 16)].astype(jnp.float32)
        plsc.store_scatter(counts_v, [rows16, tcol16], t16)
        return carry

    ccopies = []
    for ch in range(N_CHUNK):
        lax.fori_loop(ch * grp_per_chunk, (ch + 1) * grp_per_chunk, grp, 0)
        r0 = ch * rows_per_chunk
        ccopies.append(pltpu.async_copy(
            counts_v.at[pl.ds(r0, rows_per_chunk)],
            counts_hbm.at[pl.ds(base + r0, rows_per_chunk)], csem))

    for cp in ccopies:
        cp.wait()


def _tc_body(counts_ref, ttab_ref, stab_ref, out_ref):
    rt = counts_ref.shape[0]
    counts = counts_ref[...]
    tcol = counts[:, T_COL:T_COL + 1].astype(jnp.int32)
    oh_t = (tcol == lax.broadcasted_iota(jnp.int32, (rt, T_ROWS), 1)
            ).astype(jnp.bfloat16)
    tpart = jnp.dot(oh_t, ttab_ref[...], preferred_element_type=jnp.float32)
    styles = jnp.dot(counts.astype(jnp.bfloat16), stab_ref[...],
                     preferred_element_type=jnp.float32)
    out_ref[...] = jnp.concatenate([tpart, styles], axis=1)


@jax.jit
def kernel(t, genres, t_table, style_table):
    t1 = t.astype(jnp.int32)
    genres_t = genres.astype(jnp.int32).T
    ttab = jnp.zeros((T_ROWS, D), jnp.bfloat16).at[: t_table.shape[0]].set(
        t_table.astype(jnp.bfloat16))
    stab = jnp.zeros((C_PAD, D), jnp.bfloat16).at[: style_table.shape[0]].set(
        style_table.astype(jnp.bfloat16))

    mesh = plsc.VectorSubcoreMesh(core_axis_name="c", subcore_axis_name="s")
    counts = pl.kernel(
        _sc_body,
        mesh=mesh,
        compiler_params=pltpu.CompilerParams(
            needs_layout_passes=False, use_tc_tiling_on_sc=False),
        out_type=jax.ShapeDtypeStruct((B, C_PAD), jnp.float32),
        scratch_types=[
            pltpu.VMEM((RB,), jnp.int32),
            pltpu.VMEM((G, RB), jnp.int32),
            pltpu.VMEM((RB, C_PAD), jnp.float32),
            pltpu.SemaphoreType.DMA,
            pltpu.SemaphoreType.DMA,
        ],
    )(t1, genres_t)

    rt = 2048
    out = pl.pallas_call(
        _tc_body,
        grid=(B // rt,),
        in_specs=[
            pl.BlockSpec((rt, C_PAD), lambda i: (i, 0)),
            pl.BlockSpec((T_ROWS, D), lambda i: (0, 0)),
            pl.BlockSpec((C_PAD, D), lambda i: (0, 0)),
        ],
        out_specs=pl.BlockSpec((rt, 128), lambda i: (i, 0)),
        out_shape=jax.ShapeDtypeStruct((B, 128), jnp.float32),
    )(counts, ttab, stab)
    return out[:, :, None]
